# Initial kernel scaffold; baseline (speedup 1.0000x reference)
import jax
import jax.numpy as jnp
from jax.experimental import pallas as pl


def kernel(x, fwd_edges_index, bwd_edges_index, edge_attr, params):
    raise NotImplementedError("write your pallas kernel here")


# speedup vs baseline: 3.2861x; 3.2861x over previous
import functools

import jax
import jax.numpy as jnp
from jax import lax
from jax.experimental import pallas as pl
from jax.experimental.pallas import tpu as pltpu
from jax.experimental.pallas import tpu_sc as plsc

NC = 2
NS = 16
NW = NC * NS
K = 40
LANES = 16


def _edge_lin_body(ea_ref, ewf_ref, ebf_ref, ewb_ref, ebb_ref, elf_ref, elb_ref):
    ea = ea_ref[...]
    elf_ref[...] = (
        jnp.dot(ea, ewf_ref[...], preferred_element_type=jnp.float32) + ebf_ref[...]
    )
    elb_ref[...] = (
        jnp.dot(ea, ewb_ref[...], preferred_element_type=jnp.float32) + ebb_ref[...]
    )


def _edge_lin(ea, ewf, ebf, ewb, ebb):
    E, ED = ea.shape
    D = ewf.shape[1]
    BE = 8000
    return pl.pallas_call(
        _edge_lin_body,
        grid=(E // BE,),
        in_specs=[
            pl.BlockSpec((BE, ED), lambda i: (i, 0)),
            pl.BlockSpec((ED, D), lambda i: (0, 0)),
            pl.BlockSpec((1, D), lambda i: (0, 0)),
            pl.BlockSpec((ED, D), lambda i: (0, 0)),
            pl.BlockSpec((1, D), lambda i: (0, 0)),
        ],
        out_specs=[
            pl.BlockSpec((BE, D), lambda i: (i, 0)),
            pl.BlockSpec((BE, D), lambda i: (i, 0)),
        ],
        out_shape=[jax.ShapeDtypeStruct((E, D), jnp.float32)] * 2,
    )(ea, ewf.reshape(ED, D), ebf.reshape(1, D), ewb.reshape(ED, D), ebb.reshape(1, D))


def _sc_aggr_body(n_nodes, d, nch, ep,
                  x_hbm, el_hbm, src_hbm, dst_hbm, out_hbm,
                  srcb, dstb, dbuf, zb, aggr,
                  sem_el, sem_g, sem_s, sem_si, sem_di):
    c = lax.axis_index("c")
    s = lax.axis_index("s")
    wid = c * NS + s
    ebase = wid * ep

    zr = zb.shape[0]
    zero = jnp.zeros((LANES,), jnp.float32)

    @pl.loop(0, zr)
    def _(r):
        for g in range(d // LANES):
            zb[r, pl.ds(g * LANES, LANES)] = zero

    rpt = (n_nodes // NS) // 8 * 8
    rem = n_nodes - rpt * NS
    for r0 in range(0, rpt, zr):
        step = min(zr, rpt - r0)
        pltpu.sync_copy(zb.at[pl.ds(0, step)],
                        aggr.at[pl.ds(s * rpt + r0, step)])

    @pl.when(s == 0)
    def _():
        pltpu.sync_copy(zb.at[pl.ds(0, rem)],
                        aggr.at[pl.ds(rpt * NS, rem)])
    plsc.subcore_barrier()

    def el_slice(ch):
        return el_hbm.at[pl.ds(ebase + ch * K, K)]

    def src_slice(ch):
        return src_hbm.at[pl.ds(ebase + ch * K, K)]

    def dst_slice(ch):
        return dst_hbm.at[pl.ds(ebase + ch * K, K)]

    def start_loads(ch, b):
        pltpu.async_copy(src_slice(ch), srcb.at[b], sem_si.at[b])
        pltpu.async_copy(dst_slice(ch), dstb.at[b], sem_di.at[b])
        pltpu.async_copy(el_slice(ch), dbuf.at[b], sem_el.at[b])

    start_loads(0, 0)

    @pl.loop(0, nch, step=2)
    def _(ch0):
        for b in range(2):
            ch = ch0 + b
            nb = 1 - b
            pltpu.make_async_copy(el_slice(ch), dbuf.at[b], sem_el.at[b]).wait()
            pltpu.make_async_copy(src_slice(ch), srcb.at[b], sem_si.at[b]).wait()
            pltpu.make_async_copy(dst_slice(ch), dstb.at[b], sem_di.at[b]).wait()
            pltpu.async_copy(x_hbm.at[srcb.at[b]], dbuf.at[b], sem_g.at[b],
                             add=True)

            @pl.when(ch >= 1)
            def _():
                pltpu.make_async_copy(
                    dbuf.at[nb], aggr.at[dstb.at[nb]], sem_s.at[nb]).wait()

            @pl.when(ch + 1 < nch)
            def _():
                start_loads(ch + 1, nb)

            pltpu.make_async_copy(x_hbm.at[srcb.at[b]], dbuf.at[b],
                                  sem_g.at[b]).wait()

            @pl.loop(0, K)
            def _(r):
                for g in range(d // LANES):
                    sl = pl.ds(g * LANES, LANES)
                    dbuf[b, r, sl] = jnp.maximum(dbuf[b, r, sl], 0.0)

            pltpu.async_copy(dbuf.at[b], aggr.at[dstb.at[b]], sem_s.at[b],
                             add=True)

    lastb = (nch - 1) % 2
    pltpu.make_async_copy(dbuf.at[lastb], aggr.at[dstb.at[lastb]],
                          sem_s.at[lastb]).wait()
    plsc.subcore_barrier()

    for r0 in range(0, rpt, zr):
        step = min(zr, rpt - r0)
        pltpu.sync_copy(aggr.at[pl.ds(s * rpt + r0, step)],
                        zb.at[pl.ds(0, step)])
        pltpu.sync_copy(zb.at[pl.ds(0, step)],
                        out_hbm.at[c, pl.ds(s * rpt + r0, step)])

    @pl.when(s == 0)
    def _():
        pltpu.sync_copy(aggr.at[pl.ds(rpt * NS, rem)], zb.at[pl.ds(0, rem)])
        pltpu.sync_copy(zb.at[pl.ds(0, rem)],
                        out_hbm.at[c, pl.ds(rpt * NS, rem)])


def _sc_aggr(x, el, src1, dst1):
    n_nodes, d = x.shape
    ep = src1.shape[0] // NW
    nch = ep // K
    mesh = plsc.VectorSubcoreMesh(core_axis_name="c", subcore_axis_name="s")
    zr = 104
    kern = pl.kernel(
        functools.partial(_sc_aggr_body, n_nodes, d, nch, ep),
        out_type=jax.ShapeDtypeStruct((NC, n_nodes, d), jnp.float32),
        mesh=mesh,
        scratch_types=[
            pltpu.VMEM((2, K), jnp.int32),
            pltpu.VMEM((2, K), jnp.int32),
            pltpu.VMEM((2, K, d), jnp.float32),
            pltpu.VMEM((zr, d), jnp.float32),
            pltpu.VMEM_SHARED((n_nodes, d), jnp.float32),
            pltpu.SemaphoreType.DMA((2,)),
            pltpu.SemaphoreType.DMA((2,)),
            pltpu.SemaphoreType.DMA((2,)),
            pltpu.SemaphoreType.DMA((2,)),
            pltpu.SemaphoreType.DMA((2,)),
        ],
    )
    return kern(x, el, src1, dst1)


def _dense_body(x_ref, pf0, pf1, pb0, pb1,
                w1f, b1f, w2f, b2f, w1b, b1b, w2b, b2b, mwf, mwb, mb,
                h_ref, stat_ref):
    x = x_ref[...]
    hf = x + pf0[0] + pf1[0]
    hf = jnp.maximum(jnp.dot(hf, w1f[...], preferred_element_type=jnp.float32)
                     + b1f[...], 0.0)
    hf = jnp.dot(hf, w2f[...], preferred_element_type=jnp.float32) + b2f[...]
    hb = x + pb0[0] + pb1[0]
    hb = jnp.maximum(jnp.dot(hb, w1b[...], preferred_element_type=jnp.float32)
                     + b1b[...], 0.0)
    hb = jnp.dot(hb, w2b[...], preferred_element_type=jnp.float32) + b2b[...]
    hm = (jnp.dot(hf, mwf[...], preferred_element_type=jnp.float32)
          + jnp.dot(hb, mwb[...], preferred_element_type=jnp.float32) + mb[...])
    h_ref[...] = hm

    @pl.when(pl.program_id(0) == 0)
    def _():
        stat_ref[...] = jnp.zeros_like(stat_ref)

    stat_ref[0:1, :] += jnp.sum(hm, axis=0, keepdims=True)
    stat_ref[1:2, :] += jnp.sum(hm * hm, axis=0, keepdims=True)


def _dense(x, pf, pb, p):
    n_nodes, d = x.shape
    BN = 2000
    full = lambda shp: pl.BlockSpec(shp, lambda i: tuple(0 for _ in shp))
    row_blk = pl.BlockSpec((BN, d), lambda i: (i, 0))
    part0 = pl.BlockSpec((1, BN, d), lambda i: (0, i, 0))
    part1 = pl.BlockSpec((1, BN, d), lambda i: (1, i, 0))
    mwf = p['mw'][:d, :]
    mwb = p['mw'][d:, :]
    return pl.pallas_call(
        _dense_body,
        grid=(n_nodes // BN,),
        in_specs=[row_blk, part0, part1, part0, part1,
                  full((d, d)), full((1, d)), full((d, d)), full((1, d)),
                  full((d, d)), full((1, d)), full((d, d)), full((1, d)),
                  full((d, d)), full((d, d)), full((1, d))],
        out_specs=[row_blk, pl.BlockSpec((2, d), lambda i: (0, 0))],
        out_shape=[jax.ShapeDtypeStruct((n_nodes, d), jnp.float32),
                   jax.ShapeDtypeStruct((2, d), jnp.float32)],
    )(x, pf, pf, pb, pb,
      p['fwd']['w1'], p['fwd']['b1'].reshape(1, d),
      p['fwd']['w2'], p['fwd']['b2'].reshape(1, d),
      p['bwd']['w1'], p['bwd']['b1'].reshape(1, d),
      p['bwd']['w2'], p['bwd']['b2'].reshape(1, d),
      mwf, mwb, p['mb'].reshape(1, d))


def _bn_body(n_nodes, h_ref, stat_ref, g_ref, bt_ref, o_ref):
    mu = stat_ref[0:1, :] / n_nodes
    ex2 = stat_ref[1:2, :] / n_nodes
    var = ex2 - mu * mu
    inv = jax.lax.rsqrt(var + 1e-5)
    y = (h_ref[...] - mu) * inv * g_ref[...] + bt_ref[...]
    o_ref[...] = jnp.where(y > 0, y, 0.01 * y)


def _bn_leaky(h, stat, g, bt):
    n_nodes, d = h.shape
    BN = 2000
    row_blk = pl.BlockSpec((BN, d), lambda i: (i, 0))
    full = lambda shp: pl.BlockSpec(shp, lambda i: tuple(0 for _ in shp))
    return pl.pallas_call(
        functools.partial(_bn_body, n_nodes),
        grid=(n_nodes // BN,),
        in_specs=[row_blk, full((2, d)), full((1, d)), full((1, d))],
        out_specs=row_blk,
        out_shape=jax.ShapeDtypeStruct((n_nodes, d), jnp.float32),
    )(h, stat, g.reshape(1, d), bt.reshape(1, d))


def kernel(x, fwd_edges_index, bwd_edges_index, edge_attr, params):
    fsrc, fdst = fwd_edges_index[0], fwd_edges_index[1]
    bsrc, bdst = bwd_edges_index[0], bwd_edges_index[1]

    for p in params:
        elf, elb = _edge_lin(edge_attr, p['fwd']['ew'], p['fwd']['eb'],
                             p['bwd']['ew'], p['bwd']['eb'])
        pf = _sc_aggr(x, elf, fsrc, fdst)
        pb = _sc_aggr(x, elb, bsrc, bdst)
        h, stat = _dense(x, pf, pb, p)
        x = _bn_leaky(h, stat, p['g'], p['bt'])
    return x


# speedup vs baseline: 4.0710x; 1.2389x over previous
import functools

import jax
import jax.numpy as jnp
from jax import lax
from jax.experimental import pallas as pl
from jax.experimental.pallas import tpu as pltpu
from jax.experimental.pallas import tpu_sc as plsc

NC = 2
NS = 16
NW = NC * NS
K = 80
LANES = 16


def _edge_lin_body(ea_ref, ewf_ref, ebf_ref, ewb_ref, ebb_ref, elf_ref, elb_ref):
    ea = ea_ref[...]
    elf_ref[...] = (
        jnp.dot(ea, ewf_ref[...], preferred_element_type=jnp.float32) + ebf_ref[...]
    )
    elb_ref[...] = (
        jnp.dot(ea, ewb_ref[...], preferred_element_type=jnp.float32) + ebb_ref[...]
    )


def _edge_lin(ea, ewf, ebf, ewb, ebb):
    E, ED = ea.shape
    D = ewf.shape[1]
    BE = 8000
    return pl.pallas_call(
        _edge_lin_body,
        grid=(E // BE,),
        in_specs=[
            pl.BlockSpec((BE, ED), lambda i: (i, 0)),
            pl.BlockSpec((ED, D), lambda i: (0, 0)),
            pl.BlockSpec((1, D), lambda i: (0, 0)),
            pl.BlockSpec((ED, D), lambda i: (0, 0)),
            pl.BlockSpec((1, D), lambda i: (0, 0)),
        ],
        out_specs=[
            pl.BlockSpec((BE, D), lambda i: (i, 0)),
            pl.BlockSpec((BE, D), lambda i: (i, 0)),
        ],
        out_shape=[jax.ShapeDtypeStruct((E, D), jnp.float32)] * 2,
    )(ea, ewf.reshape(ED, D), ebf.reshape(1, D), ewb.reshape(ED, D), ebb.reshape(1, D))


def _sc_aggr_body(n_nodes, d, nch, ep,
                  x_hbm, el_hbm, src_hbm, dst_hbm, out_hbm,
                  srcb, dstb, dbuf, zb, aggr,
                  sem_el, sem_g, sem_s, sem_si, sem_di):
    c = lax.axis_index("c")
    s = lax.axis_index("s")
    wid = c * NS + s
    ebase = wid * ep

    zr = zb.shape[0]
    zero = jnp.zeros((LANES,), jnp.float32)

    @pl.loop(0, zr)
    def _(r):
        for g in range(d // LANES):
            zb[r, pl.ds(g * LANES, LANES)] = zero

    rpt = (n_nodes // NS) // 8 * 8
    rem = n_nodes - rpt * NS
    for r0 in range(0, rpt, zr):
        step = min(zr, rpt - r0)
        pltpu.sync_copy(zb.at[pl.ds(0, step)],
                        aggr.at[pl.ds(s * rpt + r0, step)])

    @pl.when(s == 0)
    def _():
        pltpu.sync_copy(zb.at[pl.ds(0, rem)],
                        aggr.at[pl.ds(rpt * NS, rem)])
    plsc.subcore_barrier()

    def el_slice(ch):
        return el_hbm.at[pl.ds(ebase + ch * K, K)]

    def src_slice(ch):
        return src_hbm.at[pl.ds(ebase + ch * K, K)]

    def dst_slice(ch):
        return dst_hbm.at[pl.ds(ebase + ch * K, K)]

    def start_loads(ch, b):
        pltpu.async_copy(src_slice(ch), srcb.at[b], sem_si.at[b])
        pltpu.async_copy(dst_slice(ch), dstb.at[b], sem_di.at[b])
        pltpu.async_copy(el_slice(ch), dbuf.at[b], sem_el.at[b])

    def do_chunk(ch, b, nb, first, last):
        pltpu.make_async_copy(el_slice(ch), dbuf.at[b], sem_el.at[b]).wait()
        pltpu.make_async_copy(src_slice(ch), srcb.at[b], sem_si.at[b]).wait()
        pltpu.make_async_copy(dst_slice(ch), dstb.at[b], sem_di.at[b]).wait()
        pltpu.async_copy(x_hbm.at[srcb.at[b]], dbuf.at[b], sem_g.at[b],
                         add=True)

        if not first:
            @pl.when(ch >= 1)
            def _():
                pltpu.make_async_copy(
                    dbuf.at[nb], aggr.at[dstb.at[nb]], sem_s.at[nb]).wait()

        if not last:
            @pl.when(ch + 1 < nch)
            def _():
                start_loads(ch + 1, nb)

        pltpu.make_async_copy(x_hbm.at[srcb.at[b]], dbuf.at[b],
                              sem_g.at[b]).wait()

        @pl.loop(0, K)
        def _(r):
            for g in range(d // LANES):
                sl = pl.ds(g * LANES, LANES)
                dbuf[b, r, sl] = jnp.maximum(dbuf[b, r, sl], 0.0)

        pltpu.async_copy(dbuf.at[b], aggr.at[dstb.at[b]], sem_s.at[b],
                         add=True)

    start_loads(0, 0)

    main = nch - (nch % 2)

    @pl.loop(0, main, step=2)
    def _(ch0):
        for b in range(2):
            do_chunk(ch0 + b, b, 1 - b, first=False, last=False)

    if nch % 2:
        do_chunk(nch - 1, 0, 1, first=False, last=True)
        lastb = 0
    else:
        lastb = 1
    pltpu.make_async_copy(dbuf.at[lastb], aggr.at[dstb.at[lastb]],
                          sem_s.at[lastb]).wait()
    plsc.subcore_barrier()

    for r0 in range(0, rpt, zr):
        step = min(zr, rpt - r0)
        pltpu.sync_copy(aggr.at[pl.ds(s * rpt + r0, step)],
                        zb.at[pl.ds(0, step)])
        pltpu.sync_copy(zb.at[pl.ds(0, step)],
                        out_hbm.at[c, pl.ds(s * rpt + r0, step)])

    @pl.when(s == 0)
    def _():
        pltpu.sync_copy(aggr.at[pl.ds(rpt * NS, rem)], zb.at[pl.ds(0, rem)])
        pltpu.sync_copy(zb.at[pl.ds(0, rem)],
                        out_hbm.at[c, pl.ds(rpt * NS, rem)])


def _sc_aggr(x, el, src1, dst1):
    n_nodes, d = x.shape
    ep = src1.shape[0] // NW
    nch = ep // K
    mesh = plsc.VectorSubcoreMesh(core_axis_name="c", subcore_axis_name="s")
    zr = 104
    kern = pl.kernel(
        functools.partial(_sc_aggr_body, n_nodes, d, nch, ep),
        out_type=jax.ShapeDtypeStruct((NC, n_nodes, d), jnp.float32),
        mesh=mesh,
        scratch_types=[
            pltpu.VMEM((2, K), jnp.int32),
            pltpu.VMEM((2, K), jnp.int32),
            pltpu.VMEM((2, K, d), jnp.float32),
            pltpu.VMEM((zr, d), jnp.float32),
            pltpu.VMEM_SHARED((n_nodes, d), jnp.float32),
            pltpu.SemaphoreType.DMA((2,)),
            pltpu.SemaphoreType.DMA((2,)),
            pltpu.SemaphoreType.DMA((2,)),
            pltpu.SemaphoreType.DMA((2,)),
            pltpu.SemaphoreType.DMA((2,)),
        ],
    )
    return kern(x, el, src1, dst1)


def _dense_body(x_ref, pf0, pf1, pb0, pb1,
                w1f, b1f, w2f, b2f, w1b, b1b, w2b, b2b, mwf, mwb, mb,
                h_ref, stat_ref):
    x = x_ref[...]
    hf = x + pf0[0] + pf1[0]
    hf = jnp.maximum(jnp.dot(hf, w1f[...], preferred_element_type=jnp.float32)
                     + b1f[...], 0.0)
    hf = jnp.dot(hf, w2f[...], preferred_element_type=jnp.float32) + b2f[...]
    hb = x + pb0[0] + pb1[0]
    hb = jnp.maximum(jnp.dot(hb, w1b[...], preferred_element_type=jnp.float32)
                     + b1b[...], 0.0)
    hb = jnp.dot(hb, w2b[...], preferred_element_type=jnp.float32) + b2b[...]
    hm = (jnp.dot(hf, mwf[...], preferred_element_type=jnp.float32)
          + jnp.dot(hb, mwb[...], preferred_element_type=jnp.float32) + mb[...])
    h_ref[...] = hm

    @pl.when(pl.program_id(0) == 0)
    def _():
        stat_ref[...] = jnp.zeros_like(stat_ref)

    stat_ref[0:1, :] += jnp.sum(hm, axis=0, keepdims=True)
    stat_ref[1:2, :] += jnp.sum(hm * hm, axis=0, keepdims=True)


def _dense(x, pf, pb, p):
    n_nodes, d = x.shape
    BN = 2000
    full = lambda shp: pl.BlockSpec(shp, lambda i: tuple(0 for _ in shp))
    row_blk = pl.BlockSpec((BN, d), lambda i: (i, 0))
    part0 = pl.BlockSpec((1, BN, d), lambda i: (0, i, 0))
    part1 = pl.BlockSpec((1, BN, d), lambda i: (1, i, 0))
    mwf = p['mw'][:d, :]
    mwb = p['mw'][d:, :]
    return pl.pallas_call(
        _dense_body,
        grid=(n_nodes // BN,),
        in_specs=[row_blk, part0, part1, part0, part1,
                  full((d, d)), full((1, d)), full((d, d)), full((1, d)),
                  full((d, d)), full((1, d)), full((d, d)), full((1, d)),
                  full((d, d)), full((d, d)), full((1, d))],
        out_specs=[row_blk, pl.BlockSpec((2, d), lambda i: (0, 0))],
        out_shape=[jax.ShapeDtypeStruct((n_nodes, d), jnp.float32),
                   jax.ShapeDtypeStruct((2, d), jnp.float32)],
    )(x, pf, pf, pb, pb,
      p['fwd']['w1'], p['fwd']['b1'].reshape(1, d),
      p['fwd']['w2'], p['fwd']['b2'].reshape(1, d),
      p['bwd']['w1'], p['bwd']['b1'].reshape(1, d),
      p['bwd']['w2'], p['bwd']['b2'].reshape(1, d),
      mwf, mwb, p['mb'].reshape(1, d))


def _bn_body(n_nodes, h_ref, stat_ref, g_ref, bt_ref, o_ref):
    mu = stat_ref[0:1, :] / n_nodes
    ex2 = stat_ref[1:2, :] / n_nodes
    var = ex2 - mu * mu
    inv = jax.lax.rsqrt(var + 1e-5)
    y = (h_ref[...] - mu) * inv * g_ref[...] + bt_ref[...]
    o_ref[...] = jnp.where(y > 0, y, 0.01 * y)


def _bn_leaky(h, stat, g, bt):
    n_nodes, d = h.shape
    BN = 2000
    row_blk = pl.BlockSpec((BN, d), lambda i: (i, 0))
    full = lambda shp: pl.BlockSpec(shp, lambda i: tuple(0 for _ in shp))
    return pl.pallas_call(
        functools.partial(_bn_body, n_nodes),
        grid=(n_nodes // BN,),
        in_specs=[row_blk, full((2, d)), full((1, d)), full((1, d))],
        out_specs=row_blk,
        out_shape=jax.ShapeDtypeStruct((n_nodes, d), jnp.float32),
    )(h, stat, g.reshape(1, d), bt.reshape(1, d))


def kernel(x, fwd_edges_index, bwd_edges_index, edge_attr, params):
    fsrc, fdst = fwd_edges_index[0], fwd_edges_index[1]
    bsrc, bdst = bwd_edges_index[0], bwd_edges_index[1]

    for p in params:
        elf, elb = _edge_lin(edge_attr, p['fwd']['ew'], p['fwd']['eb'],
                             p['bwd']['ew'], p['bwd']['eb'])
        pf = _sc_aggr(x, elf, fsrc, fdst)
        pb = _sc_aggr(x, elb, bsrc, bdst)
        h, stat = _dense(x, pf, pb, p)
        x = _bn_leaky(h, stat, p['g'], p['bt'])
    return x


# speedup vs baseline: 4.3497x; 1.0685x over previous
import functools

import jax
import jax.numpy as jnp
from jax import lax
from jax.experimental import pallas as pl
from jax.experimental.pallas import tpu as pltpu
from jax.experimental.pallas import tpu_sc as plsc

NC = 2
NS = 16
NW = NC * NS
K = 80
NBUF = 3
LANES = 16


def _edge_lin_body(ea_ref, ewf_ref, ebf_ref, ewb_ref, ebb_ref, elf_ref, elb_ref):
    ea = ea_ref[...]
    elf_ref[...] = (
        jnp.dot(ea, ewf_ref[...], preferred_element_type=jnp.float32) + ebf_ref[...]
    )
    elb_ref[...] = (
        jnp.dot(ea, ewb_ref[...], preferred_element_type=jnp.float32) + ebb_ref[...]
    )


def _edge_lin(ea, ewf, ebf, ewb, ebb):
    E, ED = ea.shape
    D = ewf.shape[1]
    BE = 8000
    return pl.pallas_call(
        _edge_lin_body,
        grid=(E // BE,),
        in_specs=[
            pl.BlockSpec((BE, ED), lambda i: (i, 0)),
            pl.BlockSpec((ED, D), lambda i: (0, 0)),
            pl.BlockSpec((1, D), lambda i: (0, 0)),
            pl.BlockSpec((ED, D), lambda i: (0, 0)),
            pl.BlockSpec((1, D), lambda i: (0, 0)),
        ],
        out_specs=[
            pl.BlockSpec((BE, D), lambda i: (i, 0)),
            pl.BlockSpec((BE, D), lambda i: (i, 0)),
        ],
        out_shape=[jax.ShapeDtypeStruct((E, D), jnp.float32)] * 2,
    )(ea, ewf.reshape(ED, D), ebf.reshape(1, D), ewb.reshape(ED, D), ebb.reshape(1, D))


def _sc_aggr_body(n_nodes, d, nch, ep,
                  x_hbm, el_hbm, src_hbm, dst_hbm, out_hbm,
                  srcb, dstb, dbuf, zb, aggr,
                  sem_el, sem_g, sem_s, sem_si, sem_di):
    c = lax.axis_index("c")
    s = lax.axis_index("s")
    wid = c * NS + s
    ebase = wid * ep

    zr = zb.shape[0]
    zero = jnp.zeros((LANES,), jnp.float32)

    @pl.loop(0, zr)
    def _(r):
        for g in range(d // LANES):
            zb[r, pl.ds(g * LANES, LANES)] = zero

    rpt = (n_nodes // NS) // 8 * 8
    rem = n_nodes - rpt * NS
    for r0 in range(0, rpt, zr):
        step = min(zr, rpt - r0)
        pltpu.sync_copy(zb.at[pl.ds(0, step)],
                        aggr.at[pl.ds(s * rpt + r0, step)])

    @pl.when(s == 0)
    def _():
        pltpu.sync_copy(zb.at[pl.ds(0, rem)],
                        aggr.at[pl.ds(rpt * NS, rem)])
    plsc.subcore_barrier()

    def el_slice(ch):
        return el_hbm.at[pl.ds(ebase + ch * K, K)]

    def src_slice(ch):
        return src_hbm.at[pl.ds(ebase + ch * K, K)]

    def dst_slice(ch):
        return dst_hbm.at[pl.ds(ebase + ch * K, K)]

    def start_loads(ch, b):
        pltpu.async_copy(src_slice(ch), srcb.at[b], sem_si.at[b])
        pltpu.async_copy(dst_slice(ch), dstb.at[b], sem_di.at[b])
        pltpu.async_copy(el_slice(ch), dbuf.at[b], sem_el.at[b])

    def do_chunk(ch, b, allow_load):
        pltpu.make_async_copy(el_slice(ch), dbuf.at[b], sem_el.at[b]).wait()
        pltpu.make_async_copy(src_slice(ch), srcb.at[b], sem_si.at[b]).wait()
        pltpu.make_async_copy(dst_slice(ch), dstb.at[b], sem_di.at[b]).wait()
        pltpu.async_copy(x_hbm.at[srcb.at[b]], dbuf.at[b], sem_g.at[b],
                         add=True)

        nb = (b + NBUF - 1) % NBUF

        @pl.when(ch >= 1)
        def _():
            pltpu.make_async_copy(
                dbuf.at[nb], aggr.at[dstb.at[nb]], sem_s.at[nb]).wait()

        if allow_load:
            @pl.when(ch + NBUF - 1 < nch)
            def _():
                start_loads(ch + NBUF - 1, nb)

        pltpu.make_async_copy(x_hbm.at[srcb.at[b]], dbuf.at[b],
                              sem_g.at[b]).wait()

        @pl.loop(0, K)
        def _(r):
            for g in range(d // LANES):
                sl = pl.ds(g * LANES, LANES)
                dbuf[b, r, sl] = jnp.maximum(dbuf[b, r, sl], 0.0)

        pltpu.async_copy(dbuf.at[b], aggr.at[dstb.at[b]], sem_s.at[b],
                         add=True)

    for ch in range(NBUF - 1):
        start_loads(ch, ch)

    main = nch - (nch % NBUF)

    @pl.loop(0, main, step=NBUF)
    def _(ch0):
        for b in range(NBUF):
            do_chunk(ch0 + b, b, True)

    for t in range(main, nch):
        do_chunk(t, t % NBUF, t + NBUF - 1 < nch)

    pltpu.make_async_copy(dbuf.at[(nch - 1) % NBUF],
                          aggr.at[dstb.at[(nch - 1) % NBUF]],
                          sem_s.at[(nch - 1) % NBUF]).wait()
    plsc.subcore_barrier()

    for r0 in range(0, rpt, zr):
        step = min(zr, rpt - r0)
        pltpu.sync_copy(aggr.at[pl.ds(s * rpt + r0, step)],
                        zb.at[pl.ds(0, step)])
        pltpu.sync_copy(zb.at[pl.ds(0, step)],
                        out_hbm.at[c, pl.ds(s * rpt + r0, step)])

    @pl.when(s == 0)
    def _():
        pltpu.sync_copy(aggr.at[pl.ds(rpt * NS, rem)], zb.at[pl.ds(0, rem)])
        pltpu.sync_copy(zb.at[pl.ds(0, rem)],
                        out_hbm.at[c, pl.ds(rpt * NS, rem)])


def _sc_aggr(x, el, src1, dst1):
    n_nodes, d = x.shape
    ep = src1.shape[0] // NW
    nch = ep // K
    mesh = plsc.VectorSubcoreMesh(core_axis_name="c", subcore_axis_name="s")
    zr = 48
    kern = pl.kernel(
        functools.partial(_sc_aggr_body, n_nodes, d, nch, ep),
        out_type=jax.ShapeDtypeStruct((NC, n_nodes, d), jnp.float32),
        mesh=mesh,
        scratch_types=[
            pltpu.VMEM((NBUF, K), jnp.int32),
            pltpu.VMEM((NBUF, K), jnp.int32),
            pltpu.VMEM((NBUF, K, d), jnp.float32),
            pltpu.VMEM((zr, d), jnp.float32),
            pltpu.VMEM_SHARED((n_nodes, d), jnp.float32),
            pltpu.SemaphoreType.DMA((NBUF,)),
            pltpu.SemaphoreType.DMA((NBUF,)),
            pltpu.SemaphoreType.DMA((NBUF,)),
            pltpu.SemaphoreType.DMA((NBUF,)),
            pltpu.SemaphoreType.DMA((NBUF,)),
        ],
    )
    return kern(x, el, src1, dst1)


def _dense_body(x_ref, pf0, pf1, pb0, pb1,
                w1f, b1f, w2f, b2f, w1b, b1b, w2b, b2b, mwf, mwb, mb,
                h_ref, stat_ref):
    x = x_ref[...]
    hf = x + pf0[0] + pf1[0]
    hf = jnp.maximum(jnp.dot(hf, w1f[...], preferred_element_type=jnp.float32)
                     + b1f[...], 0.0)
    hf = jnp.dot(hf, w2f[...], preferred_element_type=jnp.float32) + b2f[...]
    hb = x + pb0[0] + pb1[0]
    hb = jnp.maximum(jnp.dot(hb, w1b[...], preferred_element_type=jnp.float32)
                     + b1b[...], 0.0)
    hb = jnp.dot(hb, w2b[...], preferred_element_type=jnp.float32) + b2b[...]
    hm = (jnp.dot(hf, mwf[...], preferred_element_type=jnp.float32)
          + jnp.dot(hb, mwb[...], preferred_element_type=jnp.float32) + mb[...])
    h_ref[...] = hm

    @pl.when(pl.program_id(0) == 0)
    def _():
        stat_ref[...] = jnp.zeros_like(stat_ref)

    stat_ref[0:1, :] += jnp.sum(hm, axis=0, keepdims=True)
    stat_ref[1:2, :] += jnp.sum(hm * hm, axis=0, keepdims=True)


def _dense(x, pf, pb, p):
    n_nodes, d = x.shape
    BN = 2000
    full = lambda shp: pl.BlockSpec(shp, lambda i: tuple(0 for _ in shp))
    row_blk = pl.BlockSpec((BN, d), lambda i: (i, 0))
    part0 = pl.BlockSpec((1, BN, d), lambda i: (0, i, 0))
    part1 = pl.BlockSpec((1, BN, d), lambda i: (1, i, 0))
    mwf = p['mw'][:d, :]
    mwb = p['mw'][d:, :]
    return pl.pallas_call(
        _dense_body,
        grid=(n_nodes // BN,),
        in_specs=[row_blk, part0, part1, part0, part1,
                  full((d, d)), full((1, d)), full((d, d)), full((1, d)),
                  full((d, d)), full((1, d)), full((d, d)), full((1, d)),
                  full((d, d)), full((d, d)), full((1, d))],
        out_specs=[row_blk, pl.BlockSpec((2, d), lambda i: (0, 0))],
        out_shape=[jax.ShapeDtypeStruct((n_nodes, d), jnp.float32),
                   jax.ShapeDtypeStruct((2, d), jnp.float32)],
    )(x, pf, pf, pb, pb,
      p['fwd']['w1'], p['fwd']['b1'].reshape(1, d),
      p['fwd']['w2'], p['fwd']['b2'].reshape(1, d),
      p['bwd']['w1'], p['bwd']['b1'].reshape(1, d),
      p['bwd']['w2'], p['bwd']['b2'].reshape(1, d),
      mwf, mwb, p['mb'].reshape(1, d))


def _bn_body(n_nodes, h_ref, stat_ref, g_ref, bt_ref, o_ref):
    mu = stat_ref[0:1, :] / n_nodes
    ex2 = stat_ref[1:2, :] / n_nodes
    var = ex2 - mu * mu
    inv = jax.lax.rsqrt(var + 1e-5)
    y = (h_ref[...] - mu) * inv * g_ref[...] + bt_ref[...]
    o_ref[...] = jnp.where(y > 0, y, 0.01 * y)


def _bn_leaky(h, stat, g, bt):
    n_nodes, d = h.shape
    BN = 2000
    row_blk = pl.BlockSpec((BN, d), lambda i: (i, 0))
    full = lambda shp: pl.BlockSpec(shp, lambda i: tuple(0 for _ in shp))
    return pl.pallas_call(
        functools.partial(_bn_body, n_nodes),
        grid=(n_nodes // BN,),
        in_specs=[row_blk, full((2, d)), full((1, d)), full((1, d))],
        out_specs=row_blk,
        out_shape=jax.ShapeDtypeStruct((n_nodes, d), jnp.float32),
    )(h, stat, g.reshape(1, d), bt.reshape(1, d))


def kernel(x, fwd_edges_index, bwd_edges_index, edge_attr, params):
    fsrc, fdst = fwd_edges_index[0], fwd_edges_index[1]
    bsrc, bdst = bwd_edges_index[0], bwd_edges_index[1]

    for p in params:
        elf, elb = _edge_lin(edge_attr, p['fwd']['ew'], p['fwd']['eb'],
                             p['bwd']['ew'], p['bwd']['eb'])
        pf = _sc_aggr(x, elf, fsrc, fdst)
        pb = _sc_aggr(x, elb, bsrc, bdst)
        h, stat = _dense(x, pf, pb, p)
        x = _bn_leaky(h, stat, p['g'], p['bt'])
    return x
